# Initial kernel scaffold; baseline (speedup 1.0000x reference)
import jax
import jax.numpy as jnp
from jax.experimental import pallas as pl


def kernel(z, codebooks):
    raise NotImplementedError("write your pallas kernel here")


# speedup vs baseline: 1.0129x; 1.0129x over previous
import functools

import jax
import jax.numpy as jnp
from jax.experimental import pallas as pl

_NUM_LAYERS = 4
_K = 1024
_D = 256
_B = 16384
_TEMP = 0.5
_BB = 512


def _rowsum_sq(t):
    t = t * t
    s = t[:, :128] + t[:, 128:]
    acc = s[:, 0:8]
    for k in range(1, 16):
        acc = acc + s[:, 8 * k:8 * (k + 1)]
    a = acc[:, :4] + acc[:, 4:]
    a = a[:, :2] + a[:, 2:]
    return a[:, :1] + a[:, 1:]


def _rvq_block(z_ref, cb_ref, quant_ref, probs_ref, ids_ref):
    r = z_ref[...]
    quant = jnp.zeros_like(r)
    iota_k = jax.lax.broadcasted_iota(jnp.int32, (_BB, _K), 1)
    for layer in range(_NUM_LAYERS):
        w = cb_ref[layer]
        r2 = _rowsum_sq(r)
        w2 = _rowsum_sq(w).reshape(1, _K)
        xw = jax.lax.dot_general(
            r, w, (((1,), (1,)), ((), ())),
            preferred_element_type=jnp.float32,
            precision=jax.lax.Precision.DEFAULT,
        )
        d2 = (r2 - 2.0 * xw) + w2
        dist = jnp.sqrt(jnp.maximum(d2, 1e-12))
        logits = dist * (-1.0 / _TEMP)
        m = jnp.max(logits, axis=1, keepdims=True)
        e = jnp.exp(logits - m)
        probs_ref[layer] = e / jnp.sum(e, axis=1, keepdims=True)
        dmin = jnp.min(dist, axis=1, keepdims=True)
        ids = jnp.min(jnp.where(dist == dmin, iota_k, _K), axis=1)
        ids_ref[layer] = ids
        onehot = (iota_k == ids[:, None]).astype(jnp.float32)
        q = jax.lax.dot_general(
            onehot, w, (((1,), (0,)), ((), ())),
            preferred_element_type=jnp.float32,
            precision=jax.lax.Precision.HIGHEST,
        )
        quant = quant + q
        r = r - q
    quant_ref[...] = quant


@functools.partial(jax.jit, static_argnames=())
def kernel(z, codebooks):
    grid = (_B // _BB,)
    quant, probs, ids = pl.pallas_call(
        _rvq_block,
        grid=grid,
        in_specs=[
            pl.BlockSpec((_BB, _D), lambda i: (i, 0)),
            pl.BlockSpec((_NUM_LAYERS, _K, _D), lambda i: (0, 0, 0)),
        ],
        out_specs=[
            pl.BlockSpec((_BB, _D), lambda i: (i, 0)),
            pl.BlockSpec((_NUM_LAYERS, _BB, _K), lambda i: (0, i, 0)),
            pl.BlockSpec((_NUM_LAYERS, _BB), lambda i: (0, i)),
        ],
        out_shape=[
            jax.ShapeDtypeStruct((_B, _D), jnp.float32),
            jax.ShapeDtypeStruct((_NUM_LAYERS, _B, _K), jnp.float32),
            jax.ShapeDtypeStruct((_NUM_LAYERS, _B), jnp.int32),
        ],
    )(z, codebooks)
    return quant, probs, ids


# speedup vs baseline: 1.9468x; 1.9220x over previous
import functools

import jax
import jax.numpy as jnp
from jax.experimental import pallas as pl

_NUM_LAYERS = 4
_K = 1024
_D = 256
_B = 16384
_TEMP = 0.5
_BB = 512


def _rowsum_sq(t):
    t = t * t
    s = t[:, :128] + t[:, 128:]
    acc = s[:, 0:8]
    for k in range(1, 16):
        acc = acc + s[:, 8 * k:8 * (k + 1)]
    a = acc[:, :4] + acc[:, 4:]
    a = a[:, :2] + a[:, 2:]
    return a[:, :1] + a[:, 1:]


def _dot_default(a, b, dims):
    return jax.lax.dot_general(a, b, (dims, ((), ())),
                               preferred_element_type=jnp.float32,
                               precision=jax.lax.Precision.DEFAULT)


def _rvq_block(z_ref, cb_ref, w2_ref, hi_ref, mid_ref, lo_ref,
               quant_ref, probs_ref, ids_ref):
    r = z_ref[...]
    quant = jnp.zeros_like(r)
    iota_k = jax.lax.broadcasted_iota(jnp.int32, (_BB, _K), 1)
    for layer in range(_NUM_LAYERS):
        w = cb_ref[layer]
        r2 = _rowsum_sq(r)
        w2 = w2_ref[layer:layer + 1, :]
        xw = _dot_default(r, w, ((1,), (1,)))
        d2 = (r2 - 2.0 * xw) + w2
        dist = jnp.sqrt(jnp.maximum(d2, 1e-12))
        logits = dist * (-1.0 / _TEMP)
        dmin = jnp.min(dist, axis=1, keepdims=True)
        m = dmin * (-1.0 / _TEMP)
        e = jnp.exp(logits - m)
        probs_ref[layer] = e * (1.0 / jnp.sum(e, axis=1, keepdims=True))
        ids = jnp.min(jnp.where(dist == dmin, iota_k, _K), axis=1)
        ids_ref[layer] = ids
        onehot = (iota_k == ids[:, None]).astype(jnp.float32)
        q = (_dot_default(onehot, hi_ref[layer], ((1,), (0,)))
             + _dot_default(onehot, mid_ref[layer], ((1,), (0,)))) \
            + _dot_default(onehot, lo_ref[layer], ((1,), (0,)))
        quant = quant + q
        r = r - q
    quant_ref[...] = quant


def _split_mantissa(w):
    mask = jnp.int32(-65536)
    as_f32 = lambda x: jax.lax.bitcast_convert_type(x, jnp.float32)
    as_i32 = lambda x: jax.lax.bitcast_convert_type(x, jnp.int32)
    hi = as_f32(as_i32(w) & mask)
    rem = w - hi
    mid = as_f32(as_i32(rem) & mask)
    lo = rem - mid
    return hi, mid, lo


@functools.partial(jax.jit, static_argnames=())
def kernel(z, codebooks):
    w2 = jnp.stack([jnp.sum(codebooks[l] * codebooks[l], axis=-1)
                    for l in range(_NUM_LAYERS)])
    hi, mid, lo = _split_mantissa(codebooks)

    grid = (_B // _BB,)
    full = lambda shape: pl.BlockSpec(shape, lambda i: tuple(0 for _ in shape))
    quant, probs, ids = pl.pallas_call(
        _rvq_block,
        grid=grid,
        in_specs=[
            pl.BlockSpec((_BB, _D), lambda i: (i, 0)),
            full((_NUM_LAYERS, _K, _D)),
            full((_NUM_LAYERS, _K)),
            full((_NUM_LAYERS, _K, _D)),
            full((_NUM_LAYERS, _K, _D)),
            full((_NUM_LAYERS, _K, _D)),
        ],
        out_specs=[
            pl.BlockSpec((_BB, _D), lambda i: (i, 0)),
            pl.BlockSpec((_NUM_LAYERS, _BB, _K), lambda i: (0, i, 0)),
            pl.BlockSpec((_NUM_LAYERS, _BB), lambda i: (0, i)),
        ],
        out_shape=[
            jax.ShapeDtypeStruct((_B, _D), jnp.float32),
            jax.ShapeDtypeStruct((_NUM_LAYERS, _B, _K), jnp.float32),
            jax.ShapeDtypeStruct((_NUM_LAYERS, _B), jnp.int32),
        ],
    )(z, codebooks, w2, hi, mid, lo)
    return quant, probs, ids
